# baseline (speedup 1.0000x reference)
import jax
import jax.numpy as jnp
from jax import lax
from jax.experimental import pallas as pl
from jax.experimental.pallas import tpu as _pltpu

_PRE = 2000
_POST = 1000
_THR = 0.7
_MINSZ = 0.001
_IMG_W = 800.0
_IMG_H = 800.0

_K = 2048
_B = 128
_NB = _K // _B
_O = 1024

_HI = lax.Precision.HIGHEST


def _body(x1c, y1c, x2c, y2c, x1r, y1r, x2r, y2r, lvc, lvr, scc, out_ref,
          keep_ref, bc1, bc2, bc3, bc4, br1, br2, br3, br4):
    f32 = jnp.float32
    cx1c = jnp.minimum(jnp.maximum(x1c[...], 0.0), _IMG_W)
    cy1c = jnp.minimum(jnp.maximum(y1c[...], 0.0), _IMG_H)
    cx2c = jnp.minimum(jnp.maximum(x2c[...], 0.0), _IMG_W)
    cy2c = jnp.minimum(jnp.maximum(y2c[...], 0.0), _IMG_H)
    cx1r = jnp.minimum(jnp.maximum(x1r[...], 0.0), _IMG_W)
    cy1r = jnp.minimum(jnp.maximum(y1r[...], 0.0), _IMG_H)
    cx2r = jnp.minimum(jnp.maximum(x2r[...], 0.0), _IMG_W)
    cy2r = jnp.minimum(jnp.maximum(y2r[...], 0.0), _IMG_H)

    valid_r = ((cx2r - cx1r >= _MINSZ) & (cy2r - cy1r >= _MINSZ)).astype(f32)

    maxc = jnp.maximum(jnp.maximum(jnp.max(cx1r), jnp.max(cy1r)),
                       jnp.maximum(jnp.max(cx2r), jnp.max(cy2r)))
    step = maxc + 1.0
    off_c = lvc[...] * step
    bc1[...] = cx1c + off_c
    bc2[...] = cy1c + off_c
    bc3[...] = cx2c + off_c
    bc4[...] = cy2c + off_c
    off_r = lvr[...] * step
    br1[...] = cx1r + off_r
    br2[...] = cy1r + off_r
    br3[...] = cx2r + off_r
    br4[...] = cy2r + off_r

    keep_ref[...] = jnp.ones((_NB, _B), f32)
    ut = (lax.broadcasted_iota(jnp.int32, (_B, _B), 1)
          > lax.broadcasted_iota(jnp.int32, (_B, _B), 0)).astype(f32)

    def outer(i, _):
        ax1 = bc1[pl.ds(i * _B, _B), :]
        ay1 = bc2[pl.ds(i * _B, _B), :]
        ax2 = bc3[pl.ds(i * _B, _B), :]
        ay2 = bc4[pl.ds(i * _B, _B), :]
        area_a = (ax2 - ax1) * (ay2 - ay1)

        def iou_vs(j):
            bx1 = br1[pl.ds(j, 1), :]
            by1 = br2[pl.ds(j, 1), :]
            bx2 = br3[pl.ds(j, 1), :]
            by2 = br4[pl.ds(j, 1), :]
            area_b = (bx2 - bx1) * (by2 - by1)
            wx = jnp.maximum(jnp.minimum(ax2, bx2) - jnp.maximum(ax1, bx1), 0.0)
            wy = jnp.maximum(jnp.minimum(ay2, by2) - jnp.maximum(ay1, by1), 0.0)
            inter = wx * wy
            return inter / ((area_a + area_b) - inter + 1e-9)

        supm = (iou_vs(i) > _THR).astype(f32) * ut
        init = keep_ref[pl.ds(i, 1), :]

        def fp_body(st):
            _, kv = st
            s = jnp.dot(kv, supm, preferred_element_type=f32)
            kv2 = jnp.where(s > 0.0, 0.0, init)
            return jnp.any(kv2 != kv), kv2

        kv = lax.while_loop(lambda st: st[0], fp_body,
                            (jnp.bool_(True), init))[1]
        keep_ref[pl.ds(i, 1), :] = kv

        def cross(j, _c):
            ind = (iou_vs(j) > _THR).astype(f32)
            s = jnp.dot(kv, ind, preferred_element_type=f32)
            rowj = keep_ref[pl.ds(j, 1), :]
            keep_ref[pl.ds(j, 1), :] = rowj * (1.0 - (s > 0.0).astype(f32))
            return 0

        lax.fori_loop(i + 1, _NB, cross, 0)
        return 0

    lax.fori_loop(0, _NB, outer, 0)

    gidx = (lax.broadcasted_iota(jnp.int32, (_NB, _B), 0) * _B
            + lax.broadcasted_iota(jnp.int32, (_NB, _B), 1))
    real = (gidx < _PRE).astype(f32)
    keep = keep_ref[...] * valid_r * real
    lost = real * (1.0 - keep)

    u128 = (lax.broadcasted_iota(jnp.int32, (_B, _B), 0)
            <= lax.broadcasted_iota(jnp.int32, (_B, _B), 1)).astype(f32)
    l16 = (lax.broadcasted_iota(jnp.int32, (_NB, _NB), 1)
           < lax.broadcasted_iota(jnp.int32, (_NB, _NB), 0)).astype(f32)
    ones_col = jnp.ones((_B, 1), f32)

    def incl_cumsum(x):
        within = jnp.dot(x, u128, preferred_element_type=f32, precision=_HI)
        rowsum = jnp.dot(x, ones_col, preferred_element_type=f32,
                         precision=_HI)
        rowoff = jnp.dot(l16, rowsum, preferred_element_type=f32,
                         precision=_HI)
        return within + rowoff

    n_keep = jnp.sum(keep)
    ck = incl_cumsum(keep)
    cl = incl_cumsum(lost)
    p = jnp.where(keep > 0.0, ck - 1.0,
                  jnp.where(lost > 0.0, n_keep + cl - 1.0, 3000.0))

    o_col = lax.broadcasted_iota(jnp.int32, (_O, 1), 0).astype(f32)
    valid_c = ((cx2c - cx1c >= _MINSZ) & (cy2c - cy1c >= _MINSZ))
    scpx_c = jnp.where(valid_c, scc[...], 0.0)

    cols = (cx1c, cy1c, cx2c, cy2c, scpx_c)
    acc = [jnp.zeros((_O, 1), f32) for _ in range(5)]
    for t in range(_NB):
        p_t = lax.slice(p, (t, 0), (t + 1, _B))
        P_t = (o_col == p_t).astype(f32)
        for c in range(5):
            d_t = lax.slice(cols[c], (t * _B, 0), ((t + 1) * _B, 1))
            acc[c] = acc[c] + jnp.dot(P_t, d_t, preferred_element_type=f32,
                                      precision=_HI)
    out_score = jnp.where(o_col < n_keep, acc[4], -jnp.inf)
    out_ref[:, 0:1] = acc[0]
    out_ref[:, 1:2] = acc[1]
    out_ref[:, 2:3] = acc[2]
    out_ref[:, 3:4] = acc[3]
    out_ref[:, 4:5] = out_score


_call = pl.pallas_call(
    _body,
    out_shape=jax.ShapeDtypeStruct((_O, 8), jnp.float32),
    scratch_shapes=[_pltpu.VMEM((_NB, _B), jnp.float32)]
    + [_pltpu.VMEM((_K, 1), jnp.float32)] * 4
    + [_pltpu.VMEM((_NB, _B), jnp.float32)] * 4,
)


@jax.jit
def kernel(boxes, scores, idxs):
    top_scores, top_idx = lax.top_k(scores, _PRE)
    b = boxes[top_idx]
    lv = idxs[top_idx].astype(jnp.float32)

    bp = jnp.pad(b, ((0, _K - _PRE), (0, 0)))
    lvp = jnp.pad(lv, (0, _K - _PRE))
    scp = jnp.pad(top_scores, (0, _K - _PRE))

    cols = [bp[:, c].reshape(_K, 1) for c in range(4)]
    rows = [bp[:, c].reshape(_NB, _B) for c in range(4)]
    out = _call(*cols, *rows, lvp.reshape(_K, 1), lvp.reshape(_NB, _B),
                scp.reshape(_K, 1))
    return out[:_POST, :5]


# speedup vs baseline: 1.2660x; 1.2660x over previous
import jax
import jax.numpy as jnp
from jax import lax
from jax.experimental import pallas as pl
from jax.experimental.pallas import tpu as pltpu

_N = 20000
_PRE = 2000
_POST = 1000
_THR = 0.7
_MINSZ = 0.001
_IMG_W = 800.0
_IMG_H = 800.0

_K = 2048
_B = 128
_NB = _K // _B


def _nms_body(x1c, y1c, x2c, y2c, x1r, y1r, x2r, y2r, keep_ref):
    keep_ref[...] = jnp.ones((_NB, _B), jnp.float32)
    ut = (lax.broadcasted_iota(jnp.int32, (_B, _B), 1)
          > lax.broadcasted_iota(jnp.int32, (_B, _B), 0)).astype(jnp.float32)

    def outer(i, _):
        ax1 = x1c[pl.ds(i * _B, _B), :]
        ay1 = y1c[pl.ds(i * _B, _B), :]
        ax2 = x2c[pl.ds(i * _B, _B), :]
        ay2 = y2c[pl.ds(i * _B, _B), :]
        area_a = (ax2 - ax1) * (ay2 - ay1)

        def iou_vs(j):
            bx1 = x1r[pl.ds(j, 1), :]
            by1 = y1r[pl.ds(j, 1), :]
            bx2 = x2r[pl.ds(j, 1), :]
            by2 = y2r[pl.ds(j, 1), :]
            area_b = (bx2 - bx1) * (by2 - by1)
            wx = jnp.maximum(jnp.minimum(ax2, bx2) - jnp.maximum(ax1, bx1), 0.0)
            wy = jnp.maximum(jnp.minimum(ay2, by2) - jnp.maximum(ay1, by1), 0.0)
            inter = wx * wy
            return inter / ((area_a + area_b) - inter + 1e-9)

        supm = (iou_vs(i) > _THR).astype(jnp.float32) * ut
        init = keep_ref[pl.ds(i, 1), :]

        def fp_cond(st):
            return st[0]

        def fp_body(st):
            _, kv = st
            s = jnp.dot(kv, supm, preferred_element_type=jnp.float32)
            kv2 = jnp.where(s > 0.0, 0.0, init)
            return jnp.any(kv2 != kv), kv2

        kv = lax.while_loop(fp_cond, fp_body, (jnp.bool_(True), init))[1]
        keep_ref[pl.ds(i, 1), :] = kv

        def cross(j, _c):
            ind = (iou_vs(j) > _THR).astype(jnp.float32)
            s = jnp.dot(kv, ind, preferred_element_type=jnp.float32)
            rowj = keep_ref[pl.ds(j, 1), :]
            keep_ref[pl.ds(j, 1), :] = rowj * (1.0 - (s > 0.0).astype(jnp.float32))
            return 0

        lax.fori_loop(i + 1, _NB, cross, 0)
        return 0

    lax.fori_loop(0, _NB, outer, 0)


_nms_call = pl.pallas_call(
    _nms_body,
    out_shape=jax.ShapeDtypeStruct((_NB, _B), jnp.float32),
)


@jax.jit
def kernel(boxes, scores, idxs):
    top_scores, top_idx = lax.top_k(scores, _PRE)
    b = boxes[top_idx]
    lv = idxs[top_idx]

    bx = jnp.clip(b[:, 0::2], 0.0, _IMG_W)
    by = jnp.clip(b[:, 1::2], 0.0, _IMG_H)
    b = jnp.stack([bx[:, 0], by[:, 0], bx[:, 1], by[:, 1]], axis=1)

    ws = b[:, 2] - b[:, 0]
    hs = b[:, 3] - b[:, 1]
    valid = (ws >= _MINSZ) & (hs >= _MINSZ)
    sc = jnp.where(valid, top_scores, -jnp.inf)

    max_coordinate = b.max()
    offsets = lv.astype(b.dtype) * (max_coordinate + 1.0)
    bn = b + offsets[:, None]
    bn = jnp.pad(bn, ((0, _K - _PRE), (0, 0)))
    cols = [bn[:, c].reshape(_K, 1) for c in range(4)]
    rows = [bn[:, c].reshape(_NB, _B) for c in range(4)]
    keep_f = _nms_call(*cols, *rows)
    keep = (keep_f.reshape(_K)[:_PRE] > 0.5) & valid

    sc_kept = jnp.where(keep, sc, -jnp.inf)
    final_scores, final_idx = lax.top_k(sc_kept, _POST)
    final_boxes = b[final_idx]
    return jnp.concatenate([final_boxes, final_scores[:, None]], axis=1)
